# baseline (speedup 1.0000x reference)
import functools

import jax
import jax.numpy as jnp
from jax import lax
from jax.experimental import pallas as pl
from jax.experimental.pallas import tpu as pltpu
from jax.experimental.pallas import tpu_sc as plsc

N = 10000
E = 320000
IN_F = 128
HID = 256
OUT_G = 128
NG = 64

HR = 5000
NPH = 5120
NTR = NPH - HR
CHUNK = 128
NCH = 2560
EPAD = NCH * CHUNK
NTILES = 16
KLOC = NCH // NTILES
RPT = NPH // NTILES
BM = 1000
NBLK = N // BM


def _make_segsum(npass):
    mesh = plsc.VectorSubcoreMesh(core_axis_name="c", subcore_axis_name="s",
                                  num_cores=2, num_subcores=16)

    @functools.partial(
        pl.kernel,
        out_type=jax.ShapeDtypeStruct((2 * npass, NPH, 128), jnp.float32),
        mesh=mesh,
        scratch_types=[
            pltpu.VMEM((KLOC, CHUNK), jnp.int32),
            pltpu.VMEM((KLOC, CHUNK), jnp.int32),
            pltpu.VMEM((16,), jnp.int32),
            pltpu.VMEM((CHUNK, 128), jnp.float32),
            pltpu.VMEM((CHUNK, 128), jnp.float32),
            pltpu.VMEM_SHARED((NPH, 128), jnp.float32),
            pltpu.SemaphoreType.DMA,
            pltpu.SemaphoreType.DMA,
        ],
    )
    def seg(table, srcs, dsts, counts, zrows, out, src_l, dst_l, cnt_l,
            buf0, buf1, acc, sem0, sem1):
        cid = lax.axis_index("c")
        sid = lax.axis_index("s")
        pltpu.sync_copy(dsts.at[cid, pl.ds(sid * KLOC, KLOC)], dst_l)
        pltpu.sync_copy(counts, cnt_l)
        cv = cnt_l[...]
        nch = jnp.where(cid == 0, cv[0], cv[1])
        kc = (nch + 15 - sid) // 16
        for q in range(npass):
            grp = cid * npass + q
            pltpu.sync_copy(zrows, buf0)
            pltpu.sync_copy(buf0, acc.at[pl.ds(sid * RPT, CHUNK)])
            pltpu.sync_copy(buf0, acc.at[pl.ds(sid * RPT + CHUNK, CHUNK)])
            pltpu.sync_copy(buf0.at[pl.ds(0, RPT - 2 * CHUNK)],
                            acc.at[pl.ds(sid * RPT + 2 * CHUNK,
                                         RPT - 2 * CHUNK)])
            pltpu.sync_copy(srcs.at[cid, pl.ds(q * NCH + sid * KLOC, KLOC)],
                            src_l)
            plsc.subcore_barrier()

            @pl.when(kc > 0)
            def _():
                pltpu.async_copy(table.at[src_l.at[0]], buf0, sem0)

            @pl.when(kc > 1)
            def _():
                pltpu.async_copy(table.at[src_l.at[1]], buf1, sem1)

            def body(j, carry):
                i0 = 2 * j
                pltpu.make_async_copy(table.at[src_l.at[i0]], buf0,
                                      sem0).wait()
                pltpu.sync_copy(buf0, acc.at[dst_l.at[i0]], add=True)

                @pl.when(i0 + 2 < kc)
                def _():
                    pltpu.async_copy(table.at[src_l.at[i0 + 2]], buf0, sem0)

                @pl.when(i0 + 1 < kc)
                def _():
                    pltpu.make_async_copy(table.at[src_l.at[i0 + 1]], buf1,
                                          sem1).wait()
                    pltpu.sync_copy(buf1, acc.at[dst_l.at[i0 + 1]],
                                    add=True)

                    @pl.when(i0 + 3 < kc)
                    def _():
                        pltpu.async_copy(table.at[src_l.at[i0 + 3]], buf1,
                                         sem1)

                return carry

            lax.fori_loop(0, (kc + 1) // 2, body, 0)
            plsc.subcore_barrier()
            pltpu.sync_copy(acc.at[pl.ds(sid * RPT, RPT)],
                            out.at[grp, pl.ds(sid * RPT, RPT)])

    return seg


_SEGSUM_CACHE = {}


def _segsum(npass, *args):
    if npass not in _SEGSUM_CACHE:
        _SEGSUM_CACHE[npass] = _make_segsum(npass)
    return _SEGSUM_CACHE[npass](*args)


def _affine1(x, W, b):
    def body(x_ref, w, b_ref, hs):
        hs[...] = (jnp.dot(x_ref[...], w[...],
                           preferred_element_type=jnp.float32) + b_ref[...])

    return pl.pallas_call(
        body,
        grid=(NBLK,),
        in_specs=[
            pl.BlockSpec((BM, IN_F), lambda i: (i, 0)),
            pl.BlockSpec((IN_F, HID), lambda i: (0, 0)),
            pl.BlockSpec((1, HID), lambda i: (0, 0)),
        ],
        out_specs=pl.BlockSpec((BM, HID), lambda i: (i, 0)),
        out_shape=jax.ShapeDtypeStruct((N, HID), jnp.float32),
    )(x, W, b)


def _affine2(u, v, W, b):
    def body(u_ref, v_ref, w0, w1, b_ref, hs):
        h = jnp.dot(u_ref[...], w0[...], preferred_element_type=jnp.float32)
        h += jnp.dot(v_ref[...], w1[...], preferred_element_type=jnp.float32)
        hs[...] = h + b_ref[...]

    return pl.pallas_call(
        body,
        grid=(NBLK,),
        in_specs=[
            pl.BlockSpec((BM, 128), lambda i: (i, 0)),
            pl.BlockSpec((BM, 128), lambda i: (i, 0)),
            pl.BlockSpec((128, HID), lambda i: (0, 0)),
            pl.BlockSpec((128, HID), lambda i: (1, 0)),
            pl.BlockSpec((1, HID), lambda i: (0, 0)),
        ],
        out_specs=pl.BlockSpec((BM, HID), lambda i: (i, 0)),
        out_shape=jax.ShapeDtypeStruct((N, HID), jnp.float32),
    )(u, v, W, W, b)


def _combine1(a1, hs, Wn):
    def body(a_ref, hs_ref, wn, ha, hb):
        h = jnp.dot(a_ref[0], wn[...], preferred_element_type=jnp.float32)
        h += hs_ref[...]
        h = jnp.maximum(h, 0.0)
        ha[...] = h[:, :128]
        hb[...] = h[:, 128:]

    return pl.pallas_call(
        body,
        grid=(NBLK,),
        in_specs=[
            pl.BlockSpec((1, BM, 128), lambda i: (i // 5, i % 5, 0)),
            pl.BlockSpec((BM, HID), lambda i: (i, 0)),
            pl.BlockSpec((IN_F, HID), lambda i: (0, 0)),
        ],
        out_specs=[pl.BlockSpec((BM, 128), lambda i: (i, 0))] * 2,
        out_shape=[jax.ShapeDtypeStruct((N, 128), jnp.float32)] * 2,
    )(a1, hs, Wn)


def _combine2(a2, hs, Wn, batch3, W1, b1, W2, b2):
    def body(a20, a21, hs_ref, wn0, wn1, bat, w1, b1_ref, w2, b2_ref, y,
             sums, cnts):
        i = pl.program_id(0)
        h = jnp.dot(a20[0], wn0[...], preferred_element_type=jnp.float32)
        h += jnp.dot(a21[0], wn1[...], preferred_element_type=jnp.float32)
        h += hs_ref[...]
        h = jnp.maximum(h, 0.0)
        bv = bat[0, 0, :]
        oh = (lax.broadcasted_iota(jnp.int32, (NG, BM), 0)
              == bv[None, :]).astype(jnp.float32)
        ps = jnp.dot(oh, h, preferred_element_type=jnp.float32)
        pc = jnp.sum(oh, axis=1, keepdims=True)

        @pl.when(i == 0)
        def _():
            sums[...] = jnp.zeros((NG, HID), jnp.float32)
            cnts[...] = jnp.zeros((NG, 1), jnp.float32)

        sums[...] += ps
        cnts[...] += pc

        @pl.when(i == NBLK - 1)
        def _():
            c = jnp.maximum(cnts[...], 1.0)
            pool = sums[...] / c
            t = jnp.dot(pool, w1[...], preferred_element_type=jnp.float32)
            t = jnp.maximum(t + b1_ref[...], 0.0)
            y[...] = jnp.dot(t, w2[...],
                             preferred_element_type=jnp.float32) + b2_ref[...]

    return pl.pallas_call(
        body,
        grid=(NBLK,),
        in_specs=[
            pl.BlockSpec((1, BM, 128), lambda i: (2 * (i // 5), i % 5, 0)),
            pl.BlockSpec((1, BM, 128), lambda i: (2 * (i // 5) + 1, i % 5, 0)),
            pl.BlockSpec((BM, HID), lambda i: (i, 0)),
            pl.BlockSpec((128, HID), lambda i: (0, 0)),
            pl.BlockSpec((128, HID), lambda i: (1, 0)),
            pl.BlockSpec((1, 1, BM), lambda i: (i, 0, 0)),
            pl.BlockSpec((HID, HID), lambda i: (0, 0)),
            pl.BlockSpec((1, HID), lambda i: (0, 0)),
            pl.BlockSpec((HID, OUT_G), lambda i: (0, 0)),
            pl.BlockSpec((1, OUT_G), lambda i: (0, 0)),
        ],
        out_specs=pl.BlockSpec((NG, OUT_G), lambda i: (0, 0)),
        out_shape=jax.ShapeDtypeStruct((NG, OUT_G), jnp.float32),
        scratch_shapes=[
            pltpu.VMEM((NG, HID), jnp.float32),
            pltpu.VMEM((NG, 1), jnp.float32),
        ],
    )(a2, a2, hs, Wn, Wn, batch3, W1, b1, W2, b2)


def kernel(x, edge_index, batch, g1_Wn, g1_Ws, g1_b, g2_Wn, g2_Ws, g2_b,
           head_W1, head_b1, head_W2, head_b2):
    src = edge_index[0].astype(jnp.int32)
    dst = edge_index[1].astype(jnp.int32)
    iota = jnp.arange(E, dtype=jnp.int32)
    maskA = dst < HR
    posA = jnp.cumsum(maskA.astype(jnp.int32)) - 1
    posB = iota - posA - 1
    nA = posA[-1] + 1
    idx = jnp.where(maskA, posA, EPAD + posB)
    vals = src * 8192 + jnp.where(maskA, dst, dst - HR)
    trash2 = HR + (jnp.arange(2 * EPAD, dtype=jnp.int32) % NTR)
    delta = jnp.zeros((2 * EPAD,), jnp.int32).at[idx].add(
        vals - (HR + idx % NTR))
    packed = trash2 + delta
    srcP = packed // 8192
    dstP = packed % 8192

    def il(a):
        return (a.reshape(KLOC, NTILES, CHUNK).transpose(1, 0, 2)
                .reshape(NCH, CHUNK))

    srcA, srcB = il(srcP[:EPAD]), il(srcP[EPAD:])
    dsts = jnp.stack([il(dstP[:EPAD]), il(dstP[EPAD:])])
    srcs1 = jnp.stack([srcA, srcB])
    srcs2 = jnp.stack([jnp.concatenate([srcA, srcA + N]),
                       jnp.concatenate([srcB, srcB + N])])
    nchA = (nA + CHUNK - 1) // CHUNK
    nchB = (E - nA + CHUNK - 1) // CHUNK
    counts = (jnp.zeros((16,), jnp.int32).at[0].set(nchA).at[1].set(nchB))
    z128 = jnp.zeros((CHUNK, 128), jnp.float32)

    hs1 = _affine1(x, g1_Ws, g1_b.reshape(1, HID))
    a1 = _segsum(1, x, srcs1, dsts, counts, z128)
    h1a, h1b = _combine1(a1, hs1, g1_Wn)
    t2 = jnp.concatenate([h1a, h1b], axis=0)
    hs2 = _affine2(h1a, h1b, g2_Ws, g2_b.reshape(1, HID))
    a2 = _segsum(2, t2, srcs2, dsts, counts, z128)
    return _combine2(a2, hs2, g2_Wn,
                     batch.astype(jnp.int32).reshape(NBLK, 1, BM),
                     head_W1, head_b1.reshape(1, HID),
                     head_W2, head_b2.reshape(1, OUT_G))


# speedup vs baseline: 1.0056x; 1.0056x over previous
import functools

import jax
import jax.numpy as jnp
from jax import lax
from jax.experimental import pallas as pl
from jax.experimental.pallas import tpu as pltpu
from jax.experimental.pallas import tpu_sc as plsc

N = 10000
E = 320000
IN_F = 128
HID = 256
OUT_G = 128
NG = 64

HR = 5000
NPH = 5120
NTR = NPH - HR
CHUNK = 128
NCH = 2560
EPAD = NCH * CHUNK
NTILES = 16
KLOC = NCH // NTILES
RPT = NPH // NTILES
BM = 1000
NBLK = N // BM


def _make_segsum(npass):
    mesh = plsc.VectorSubcoreMesh(core_axis_name="c", subcore_axis_name="s",
                                  num_cores=2, num_subcores=16)

    @functools.partial(
        pl.kernel,
        out_type=jax.ShapeDtypeStruct((2 * npass, NPH, 128), jnp.float32),
        mesh=mesh,
        scratch_types=[
            pltpu.VMEM((KLOC, CHUNK), jnp.int32),
            pltpu.VMEM((KLOC, CHUNK), jnp.int32),
            pltpu.VMEM((16,), jnp.int32),
            pltpu.VMEM((CHUNK, 128), jnp.float32),
            pltpu.VMEM((CHUNK, 128), jnp.float32),
            pltpu.VMEM_SHARED((NPH, 128), jnp.float32),
            pltpu.SemaphoreType.DMA,
            pltpu.SemaphoreType.DMA,
        ],
    )
    def seg(table, srcs, dsts, counts, zrows, out, src_l, dst_l, cnt_l,
            buf0, buf1, acc, sem0, sem1):
        cid = lax.axis_index("c")
        sid = lax.axis_index("s")
        pltpu.sync_copy(dsts.at[cid, pl.ds(sid * KLOC, KLOC)], dst_l)
        pltpu.sync_copy(counts, cnt_l)
        cv = cnt_l[...]
        nch = jnp.where(cid == 0, cv[0], cv[1])
        kc = (nch + 15 - sid) // 16
        for q in range(npass):
            grp = cid * npass + q
            pltpu.sync_copy(zrows, buf0)
            pltpu.sync_copy(buf0, acc.at[pl.ds(sid * RPT, CHUNK)])
            pltpu.sync_copy(buf0, acc.at[pl.ds(sid * RPT + CHUNK, CHUNK)])
            pltpu.sync_copy(buf0.at[pl.ds(0, RPT - 2 * CHUNK)],
                            acc.at[pl.ds(sid * RPT + 2 * CHUNK,
                                         RPT - 2 * CHUNK)])
            pltpu.sync_copy(srcs.at[cid, pl.ds(q * NCH + sid * KLOC, KLOC)],
                            src_l)
            plsc.subcore_barrier()

            @pl.when(kc > 0)
            def _():
                pltpu.async_copy(table.at[src_l.at[0]], buf0, sem0)

            @pl.when(kc > 1)
            def _():
                pltpu.async_copy(table.at[src_l.at[1]], buf1, sem1)

            def body(j, carry):
                i0 = 2 * j
                pltpu.make_async_copy(table.at[src_l.at[i0]], buf0,
                                      sem0).wait()
                pltpu.sync_copy(buf0, acc.at[dst_l.at[i0]], add=True)

                @pl.when(i0 + 2 < kc)
                def _():
                    pltpu.async_copy(table.at[src_l.at[i0 + 2]], buf0, sem0)

                @pl.when(i0 + 1 < kc)
                def _():
                    pltpu.make_async_copy(table.at[src_l.at[i0 + 1]], buf1,
                                          sem1).wait()
                    pltpu.sync_copy(buf1, acc.at[dst_l.at[i0 + 1]],
                                    add=True)

                    @pl.when(i0 + 3 < kc)
                    def _():
                        pltpu.async_copy(table.at[src_l.at[i0 + 3]], buf1,
                                         sem1)

                return carry

            lax.fori_loop(0, (kc + 1) // 2, body, 0)
            plsc.subcore_barrier()
            pltpu.sync_copy(acc.at[pl.ds(sid * RPT, RPT)],
                            out.at[grp, pl.ds(sid * RPT, RPT)])

    return seg


_SEGSUM_CACHE = {}


def _segsum(npass, *args):
    if npass not in _SEGSUM_CACHE:
        _SEGSUM_CACHE[npass] = _make_segsum(npass)
    return _SEGSUM_CACHE[npass](*args)


def _combine1(a1, x, Wn, Ws, b):
    def body(a_ref, x_ref, wn, ws, b_ref, ha, hb):
        h = jnp.dot(a_ref[0], wn[...], preferred_element_type=jnp.float32)
        h += jnp.dot(x_ref[...], ws[...], preferred_element_type=jnp.float32)
        h += b_ref[...]
        h = jnp.maximum(h, 0.0)
        ha[...] = h[:, :128]
        hb[...] = h[:, 128:]

    return pl.pallas_call(
        body,
        grid=(NBLK,),
        in_specs=[
            pl.BlockSpec((1, BM, 128), lambda i: (i // 5, i % 5, 0)),
            pl.BlockSpec((BM, IN_F), lambda i: (i, 0)),
            pl.BlockSpec((IN_F, HID), lambda i: (0, 0)),
            pl.BlockSpec((IN_F, HID), lambda i: (0, 0)),
            pl.BlockSpec((1, HID), lambda i: (0, 0)),
        ],
        out_specs=[pl.BlockSpec((BM, 128), lambda i: (i, 0))] * 2,
        out_shape=[jax.ShapeDtypeStruct((N, 128), jnp.float32)] * 2,
    )(a1, x, Wn, Ws, b)


def _combine2(a2, h1a, h1b, Wn, Ws, b, batch3):
    def body(a20, a21, ha, hb, wn0, wn1, ws0, ws1, b_ref, bat, sums, cnts):
        i = pl.program_id(0)
        h = jnp.dot(a20[0], wn0[...], preferred_element_type=jnp.float32)
        h += jnp.dot(a21[0], wn1[...], preferred_element_type=jnp.float32)
        h += jnp.dot(ha[...], ws0[...], preferred_element_type=jnp.float32)
        h += jnp.dot(hb[...], ws1[...], preferred_element_type=jnp.float32)
        h += b_ref[...]
        h = jnp.maximum(h, 0.0)
        bv = bat[0, 0, :]
        oh = (lax.broadcasted_iota(jnp.int32, (NG, BM), 0)
              == bv[None, :]).astype(jnp.float32)
        ps = jnp.dot(oh, h, preferred_element_type=jnp.float32)
        pc = jnp.sum(oh, axis=1, keepdims=True)

        @pl.when(i == 0)
        def _():
            sums[...] = jnp.zeros((NG, HID), jnp.float32)
            cnts[...] = jnp.zeros((NG, 128), jnp.float32)

        sums[...] += ps
        cnts[...] += jnp.broadcast_to(pc, (NG, 128))

    return pl.pallas_call(
        body,
        grid=(NBLK,),
        in_specs=[
            pl.BlockSpec((1, BM, 128), lambda i: (2 * (i // 5), i % 5, 0)),
            pl.BlockSpec((1, BM, 128), lambda i: (2 * (i // 5) + 1, i % 5, 0)),
            pl.BlockSpec((BM, 128), lambda i: (i, 0)),
            pl.BlockSpec((BM, 128), lambda i: (i, 0)),
            pl.BlockSpec((128, HID), lambda i: (0, 0)),
            pl.BlockSpec((128, HID), lambda i: (1, 0)),
            pl.BlockSpec((128, HID), lambda i: (0, 0)),
            pl.BlockSpec((128, HID), lambda i: (1, 0)),
            pl.BlockSpec((1, HID), lambda i: (0, 0)),
            pl.BlockSpec((1, 1, BM), lambda i: (i, 0, 0)),
        ],
        out_specs=[
            pl.BlockSpec((NG, HID), lambda i: (0, 0)),
            pl.BlockSpec((NG, 128), lambda i: (0, 0)),
        ],
        out_shape=[
            jax.ShapeDtypeStruct((NG, HID), jnp.float32),
            jax.ShapeDtypeStruct((NG, 128), jnp.float32),
        ],
    )(a2, a2, h1a, h1b, Wn, Wn, Ws, Ws, b, batch3)


def _head(sums, cnts, W1, b1, W2, b2):
    def body(s_ref, c_ref, w1, b1_ref, w2, b2_ref, y):
        c = jnp.maximum(c_ref[:, 0:1], 1.0)
        pool = s_ref[...] / c
        t = jnp.dot(pool, w1[...], preferred_element_type=jnp.float32)
        t = jnp.maximum(t + b1_ref[...], 0.0)
        y[...] = jnp.dot(t, w2[...],
                         preferred_element_type=jnp.float32) + b2_ref[...]

    return pl.pallas_call(
        body,
        out_shape=jax.ShapeDtypeStruct((NG, OUT_G), jnp.float32),
    )(sums, cnts, W1, b1, W2, b2)


def kernel(x, edge_index, batch, g1_Wn, g1_Ws, g1_b, g2_Wn, g2_Ws, g2_b,
           head_W1, head_b1, head_W2, head_b2):
    src = edge_index[0].astype(jnp.int32)
    dst = edge_index[1].astype(jnp.int32)
    iota = jnp.arange(E, dtype=jnp.int32)
    maskA = dst < HR
    posA = jnp.cumsum(maskA.astype(jnp.int32)) - 1
    posB = iota - posA - 1
    nA = posA[-1] + 1
    idx = jnp.where(maskA, posA, EPAD + posB)
    vals = src * 8192 + jnp.where(maskA, dst, dst - HR)
    trash2 = HR + (jnp.arange(2 * EPAD, dtype=jnp.int32) % NTR)
    delta = jnp.zeros((2 * EPAD,), jnp.int32).at[idx].add(
        vals - (HR + idx % NTR))
    packed = trash2 + delta
    srcP = packed // 8192
    dstP = packed % 8192

    def il(a):
        return (a.reshape(KLOC, NTILES, CHUNK).transpose(1, 0, 2)
                .reshape(NCH, CHUNK))

    srcA, srcB = il(srcP[:EPAD]), il(srcP[EPAD:])
    dsts = jnp.stack([il(dstP[:EPAD]), il(dstP[EPAD:])])
    srcs1 = jnp.stack([srcA, srcB])
    srcs2 = jnp.stack([jnp.concatenate([srcA, srcA + N]),
                       jnp.concatenate([srcB, srcB + N])])
    nchA = (nA + CHUNK - 1) // CHUNK
    nchB = (E - nA + CHUNK - 1) // CHUNK
    counts = (jnp.zeros((16,), jnp.int32).at[0].set(nchA).at[1].set(nchB))
    z128 = jnp.zeros((CHUNK, 128), jnp.float32)

    a1 = _segsum(1, x, srcs1, dsts, counts, z128)
    h1a, h1b = _combine1(a1, x, g1_Wn, g1_Ws, g1_b.reshape(1, HID))
    t2 = jnp.concatenate([h1a, h1b], axis=0)
    a2 = _segsum(2, t2, srcs2, dsts, counts, z128)
    sums, cnts = _combine2(a2, h1a, h1b, g2_Wn, g2_Ws,
                           g2_b.reshape(1, HID),
                           batch.astype(jnp.int32).reshape(NBLK, 1, BM))
    return _head(sums, cnts, head_W1, head_b1.reshape(1, HID),
                 head_W2, head_b2.reshape(1, OUT_G))
